# baseline (speedup 1.0000x reference)
import functools

import jax
import jax.numpy as jnp
from jax import lax
from jax.experimental import pallas as pl
from jax.experimental.pallas import tpu as pltpu
from jax.experimental.pallas import tpu_sc as plsc

N = 10000
E = 160000
D = 256
NC = 2
NS = 16
ER = E // 128
NPAD = N + 16

_mesh = plsc.VectorSubcoreMesh(
    core_axis_name="c", subcore_axis_name="s", num_cores=NC, num_subcores=NS)

_f32 = jnp.float32
_i32 = jnp.int32


def _sigmoid16(m):
    return 1.0 / (1.0 + jnp.exp(-m))


def _take16(vec, idx16):
    return lax.gather(
        vec, idx16[:, None],
        lax.GatherDimensionNumbers(
            offset_dims=(), collapsed_slice_dims=(0,), start_index_map=(0,)),
        (1,), mode=lax.GatherScatterMode.PROMISE_IN_BOUNDS)


@functools.partial(
    pl.kernel,
    out_type=[
        jax.ShapeDtypeStruct((2, N, 128), _f32),
        jax.ShapeDtypeStruct((N,), _f32),
        jax.ShapeDtypeStruct((N,), _f32),
        jax.ShapeDtypeStruct((N,), _f32),
    ],
    mesh=_mesh,
    scratch_types=[
        pltpu.VMEM((128,), _i32),
        pltpu.VMEM((128,), _i32),
        pltpu.VMEM((128,), _i32),
        pltpu.VMEM((128,), _i32),
        pltpu.VMEM((128,), _f32),
        pltpu.VMEM((128,), _f32),
        pltpu.VMEM((128,), _f32),
        pltpu.VMEM((128,), _f32),
        pltpu.VMEM((128,), _f32),
        pltpu.VMEM((128,), _f32),
        pltpu.VMEM((128,), _f32),
        pltpu.VMEM((128,), _f32),
        pltpu.VMEM((128,), _f32),
        pltpu.VMEM((16,), _i32),
        pltpu.VMEM((16,), _i32),
        pltpu.VMEM((16,), _f32),
        pltpu.VMEM((16, 128), _f32),
        pltpu.VMEM((16, 128), _f32),
        pltpu.VMEM((2000,), _f32),
        pltpu.VMEM((16,), _i32),
        pltpu.VMEM_SHARED((NPAD, 128), _f32),
        pltpu.VMEM_SHARED((N,), _f32),
        pltpu.VMEM_SHARED((N,), _f32),
        pltpu.VMEM_SHARED((N,), _f32),
        pltpu.VMEM_SHARED((N,), _f32),
        pltpu.SemaphoreType.DMA,
        pltpu.SemaphoreType.DMA,
        pltpu.SemaphoreType.DMA,
        pltpu.SemaphoreType.DMA,
    ],
)
def _sc_main(x2_hbm, dst2, src2, em2, st_hbm, aggT, deg_out, cs_out, ct_out,
             d_a, d_b, s_a, s_b, w_a, w_b, rcs_a, rcs_b,
             rct_a, rct_b, rmk_a, rmk_b, rm, si16, dg16, red16, xbuf,
             zrow, zbuf,
             stv, agg_s, deg_s, cs_s, ct_s, mk_s,
             gsem_a, gsem_b, ssem_a, ssem_x):
    c = lax.axis_index("c")
    sid = lax.axis_index("s")
    zv = jnp.zeros((16,), _f32)

    def _zb(i, _):
        zbuf[pl.ds(i * 16, 16)] = zv
        return 0
    lax.fori_loop(0, 125, _zb, 0)

    def _zr(i, _):
        for kk in range(8):
            zrow[i, pl.ds(kk * 16, 16)] = zv
        return 0
    lax.fori_loop(0, 16, _zr, 0)
    pltpu.sync_copy(st_hbm, stv)

    @pl.when(sid < 5)
    def _():
        pltpu.sync_copy(zbuf, deg_s.at[pl.ds(sid * 2000, 2000)])
        pltpu.sync_copy(zbuf, mk_s.at[pl.ds(sid * 2000, 2000)])

    @pl.when(jnp.logical_and(sid >= 5, sid < 10))
    def _():
        pltpu.sync_copy(zbuf, cs_s.at[pl.ds((sid - 5) * 2000, 2000)])

    @pl.when(jnp.logical_and(sid >= 10, sid < 15))
    def _():
        pltpu.sync_copy(zbuf, ct_s.at[pl.ds((sid - 10) * 2000, 2000)])

    stv_v = stv[...]
    s_vec = _take16(stv_v, jnp.zeros((16,), _i32))
    t_vec = _take16(stv_v, jnp.full((16,), 1, _i32))

    stripe = sid * 640
    zcnt = jnp.where(sid < 15, 40, 26)

    def _bz(q, _):
        pltpu.sync_copy(zrow, agg_s.at[pl.ds(stripe + q * 16, 16)])
        return 0
    lax.fori_loop(0, zcnt, _bz, 0)

    plsc.subcore_barrier()

    def _stage1(row, d_r, s_r, w_r, rcs_r, rct_r, rmk_r):
        pltpu.sync_copy(dst2.at[row], d_r)
        pltpu.sync_copy(src2.at[row], s_r)
        pltpu.sync_copy(em2.at[row], rm)

        def _grp(k, _):
            m16 = rm[pl.ds(k * 16, 16)]
            d16 = d_r[pl.ds(k * 16, 16)]
            w16 = _sigmoid16(m16)
            w_r[pl.ds(k * 16, 16)] = w16
            rcs16 = jnp.where(d16 == s_vec, w16, 0.0)
            rct16 = jnp.where(d16 == t_vec, w16, 0.0)
            rcs_r[pl.ds(k * 16, 16)] = rcs16
            rct_r[pl.ds(k * 16, 16)] = rct16
            rmk_r[pl.ds(k * 16, 16)] = rcs16 + rct16
            return 0
        lax.fori_loop(0, 8, _grp, 0)
        h1 = pltpu.async_copy(w_r, deg_s.at[d_r], ssem_a, add=True)
        h2 = pltpu.async_copy(rcs_r, cs_s.at[s_r], ssem_a, add=True)
        h3 = pltpu.async_copy(rct_r, ct_s.at[s_r], ssem_a, add=True)
        h4 = pltpu.async_copy(rmk_r, mk_s.at[s_r], ssem_a, add=True)
        return h1, h2, h3, h4

    def _pair1(gp, _):
        row0 = sid + (2 * gp) * NS
        row1 = row0 + NS
        both = row1 < ER

        @pl.when(both)
        def _():
            ha = _stage1(row0, d_a, s_a, w_a, rcs_a, rct_a, rmk_a)
            hb = _stage1(row1, d_b, s_b, w_b, rcs_b, rct_b, rmk_b)
            for h in ha + hb:
                h.wait()

        @pl.when(jnp.logical_and(row0 < ER, jnp.logical_not(both)))
        def _():
            ha = _stage1(row0, d_a, s_a, w_a, rcs_a, rct_a, rmk_a)
            for h in ha:
                h.wait()
        return 0
    lax.fori_loop(0, 40, _pair1, 0)

    plsc.subcore_barrier()

    def _stage2(row, d_r, s_r, w_r, mrow_r, gsem):
        pltpu.sync_copy(dst2.at[row], d_r)
        pltpu.sync_copy(src2.at[row], s_r)
        pltpu.sync_copy(em2.at[row], rm)

        def _grp(k, _):
            w_r[pl.ds(k * 16, 16)] = _sigmoid16(rm[pl.ds(k * 16, 16)])
            return 0
        lax.fori_loop(0, 8, _grp, 0)
        return pltpu.async_copy(mk_s.at[d_r], mrow_r, gsem)

    lane16 = lax.iota(_i32, 16)
    xor8 = jnp.bitwise_xor(lane16, 8)
    xor4 = jnp.bitwise_xor(lane16, 4)
    xor2 = jnp.bitwise_xor(lane16, 2)
    xor1 = jnp.bitwise_xor(lane16, 1)

    def _proc2(d_r, s_r, w_r, mrow_r):
        for k in range(8):
            m16 = mrow_r[pl.ds(k * 16, 16)]
            v = jnp.maximum(m16, _take16(m16, xor8))
            v = jnp.maximum(v, _take16(v, xor4))
            v = jnp.maximum(v, _take16(v, xor2))
            v = jnp.maximum(v, _take16(v, xor1))
            hit = v[0] > 0.0

            @pl.when(hit)
            def _():
                s16 = s_r[pl.ds(k * 16, 16)]
                si16[...] = s16 + s16 + c
                dg16[...] = d_r[pl.ds(k * 16, 16)]
                hg = pltpu.async_copy(x2_hbm.at[si16], xbuf, gsem_a)
                hg.wait()
                w16 = w_r[pl.ds(k * 16, 16)]
                for r in range(16):
                    wr16 = _take16(w16, jnp.full((16,), r, _i32))
                    for kk in range(8):
                        xbuf[r, pl.ds(kk * 16, 16)] = (
                            xbuf[r, pl.ds(kk * 16, 16)] * wr16)
                hs = pltpu.async_copy(xbuf, agg_s.at[dg16], ssem_x, add=True)
                hs.wait()

    def _pair2(gp, _):
        row0 = sid + (2 * gp) * NS
        row1 = row0 + NS
        both = row1 < ER

        @pl.when(both)
        def _():
            ha = _stage2(row0, d_a, s_a, w_a, rcs_a, gsem_a)
            hb = _stage2(row1, d_b, s_b, w_b, rcs_b, gsem_b)
            ha.wait()
            _proc2(d_a, s_a, w_a, rcs_a)
            hb.wait()
            _proc2(d_b, s_b, w_b, rcs_b)

        @pl.when(jnp.logical_and(row0 < ER, jnp.logical_not(both)))
        def _():
            ha = _stage2(row0, d_a, s_a, w_a, rcs_a, gsem_a)
            ha.wait()
            _proc2(d_a, s_a, w_a, rcs_a)
        return 0
    lax.fori_loop(0, 40, _pair2, 0)

    plsc.subcore_barrier()

    @pl.when(sid < 15)
    def _():
        pltpu.sync_copy(agg_s.at[pl.ds(sid * 632, 632)],
                        aggT.at[c, pl.ds(sid * 632, 632)])

    @pl.when(sid == 15)
    def _():
        pltpu.sync_copy(agg_s.at[pl.ds(9480, 520)],
                        aggT.at[c, pl.ds(9480, 520)])

    @pl.when(jnp.logical_and(c == 0, sid < 5))
    def _():
        pltpu.sync_copy(deg_s.at[pl.ds(sid * 2000, 2000)], zbuf)
        pltpu.sync_copy(zbuf, deg_out.at[pl.ds(sid * 2000, 2000)])

    @pl.when(jnp.logical_and(c == 0, jnp.logical_and(sid >= 5, sid < 10)))
    def _():
        pltpu.sync_copy(cs_s.at[pl.ds((sid - 5) * 2000, 2000)], zbuf)
        pltpu.sync_copy(zbuf, cs_out.at[pl.ds((sid - 5) * 2000, 2000)])

    @pl.when(jnp.logical_and(c == 0, jnp.logical_and(sid >= 10, sid < 15)))
    def _():
        pltpu.sync_copy(ct_s.at[pl.ds((sid - 10) * 2000, 2000)], zbuf)
        pltpu.sync_copy(zbuf, ct_out.at[pl.ds((sid - 10) * 2000, 2000)])


_BN = 1000


def _tc_h1_body(aggT_ref, deg_ref, w1_ref, h1_ref):
    degb = deg_ref[...] + 1e-9
    a0 = aggT_ref[0] / degb
    a1 = aggT_ref[1] / degb
    w1 = w1_ref[...]
    z = (jnp.dot(a0, w1[:128, :], preferred_element_type=_f32,
                 precision=lax.Precision.HIGHEST)
         + jnp.dot(a1, w1[128:, :], preferred_element_type=_f32,
                   precision=lax.Precision.HIGHEST))
    h1_ref[...] = jnp.maximum(z, 0.0)


def _tc_h1(aggT, deg2, W1):
    return pl.pallas_call(
        _tc_h1_body,
        grid=(N // _BN,),
        in_specs=[
            pl.BlockSpec((2, _BN, 128), lambda i: (0, i, 0)),
            pl.BlockSpec((_BN, 1), lambda i: (i, 0)),
            pl.BlockSpec((D, D), lambda i: (0, 0)),
        ],
        out_specs=pl.BlockSpec((_BN, D), lambda i: (i, 0)),
        out_shape=jax.ShapeDtypeStruct((N, D), _f32),
    )(aggT, deg2, W1)


def _tc_final_body(st_ref, cvec_ref, h1_ref, deg_ref, w2_ref, em_ref,
                   out_ref):
    s = st_ref[0, 0]
    t = st_ref[0, 1]
    acc = jnp.dot(cvec_ref[...], h1_ref[...], preferred_element_type=_f32,
                  precision=lax.Precision.HIGHEST)
    deg_s = deg_ref[pl.ds(s, 1), :][0, 0] + 1e-9
    deg_t = deg_ref[pl.ds(t, 1), :][0, 0] + 1e-9
    h2s = jnp.dot((acc[0, :] / deg_s).reshape(1, D), w2_ref[...],
                  preferred_element_type=_f32,
                  precision=lax.Precision.HIGHEST)
    h2t = jnp.dot((acc[1, :] / deg_t).reshape(1, D), w2_ref[...],
                  preferred_element_type=_f32,
                  precision=lax.Precision.HIGHEST)
    score = jnp.sum(h2s * h2t)
    w = jax.nn.sigmoid(em_ref[...])
    eps = 1e-6
    wc = jnp.clip(w, eps, 1.0 - eps)
    ent = -(wc * jnp.log(wc) + (1.0 - wc) * jnp.log(1.0 - wc))
    loss = (-jax.nn.log_sigmoid(score)
            + jnp.sum(w) / E + jnp.sum(ent) / E)
    out_ref[...] = jnp.reshape(loss, (1, 1))


def _tc_final(st2, cvec, h1, deg2, W2, em2):
    return pl.pallas_call(
        _tc_final_body,
        in_specs=[
            pl.BlockSpec(memory_space=pltpu.SMEM),
            pl.BlockSpec((2, N), lambda: (0, 0)),
            pl.BlockSpec((N, D), lambda: (0, 0)),
            pl.BlockSpec((N, 1), lambda: (0, 0)),
            pl.BlockSpec((D, D), lambda: (0, 0)),
            pl.BlockSpec((ER, 128), lambda: (0, 0)),
        ],
        out_specs=pl.BlockSpec((1, 1), lambda: (0, 0)),
        out_shape=jax.ShapeDtypeStruct((1, 1), _f32),
    )(st2, cvec, h1, deg2, W2, em2)


def kernel(x, edge_index, edge_mask, src_nid, tgt_nid, W1, W2):
    src = edge_index[0]
    dst = edge_index[1]
    src2 = src.reshape(ER, 128)
    dst2 = dst.reshape(ER, 128)
    em2 = edge_mask.reshape(ER, 128)
    st = jnp.zeros((16,), _i32)
    st = st.at[0].set(jnp.asarray(src_nid, _i32))
    st = st.at[1].set(jnp.asarray(tgt_nid, _i32))

    x2 = x.reshape(2 * N, 128)
    aggT, deg, cs, ct = _sc_main(x2, dst2, src2, em2, st)
    cvec = jnp.stack([cs, ct])
    deg2 = deg.reshape(N, 1)
    h1 = _tc_h1(aggT, deg2, W1)
    out = _tc_final(st[:2].reshape(1, 2), cvec, h1, deg2, W2, em2)
    return out[0, 0]


# speedup vs baseline: 1.5338x; 1.5338x over previous
import functools

import jax
import jax.numpy as jnp
from jax import lax
from jax.experimental import pallas as pl
from jax.experimental.pallas import tpu as pltpu
from jax.experimental.pallas import tpu_sc as plsc

N = 10000
E = 160000
D = 256
NC = 2
NS = 16
ER = E // 128
NPAD = N + 16

_mesh = plsc.VectorSubcoreMesh(
    core_axis_name="c", subcore_axis_name="s", num_cores=NC, num_subcores=NS)

_f32 = jnp.float32
_i32 = jnp.int32


def _sigmoid16(m):
    return 1.0 / (1.0 + jnp.exp(-m))


def _take16(vec, idx16):
    return lax.gather(
        vec, idx16[:, None],
        lax.GatherDimensionNumbers(
            offset_dims=(), collapsed_slice_dims=(0,), start_index_map=(0,)),
        (1,), mode=lax.GatherScatterMode.PROMISE_IN_BOUNDS)


@functools.partial(
    pl.kernel,
    out_type=[
        jax.ShapeDtypeStruct((2, N, 128), _f32),
        jax.ShapeDtypeStruct((N,), _f32),
        jax.ShapeDtypeStruct((N,), _f32),
        jax.ShapeDtypeStruct((N,), _f32),
    ],
    mesh=_mesh,
    scratch_types=[
        pltpu.VMEM((256,), _i32),
        pltpu.VMEM((256,), _i32),
        pltpu.VMEM((128,), _i32),
        pltpu.VMEM((128,), _i32),
        pltpu.VMEM((128,), _f32),
        pltpu.VMEM((128,), _f32),
        pltpu.VMEM((128,), _f32),
        pltpu.VMEM((128,), _f32),
        pltpu.VMEM((128,), _f32),
        pltpu.VMEM((128,), _f32),
        pltpu.VMEM((128,), _f32),
        pltpu.VMEM((128, 128), _f32),
        pltpu.VMEM((128, 128), _f32),
        pltpu.VMEM((16, 128), _f32),
        pltpu.VMEM((2000,), _f32),
        pltpu.VMEM((16,), _i32),
        pltpu.VMEM_SHARED((NPAD, 128), _f32),
        pltpu.VMEM_SHARED((N,), _f32),
        pltpu.VMEM_SHARED((N,), _f32),
        pltpu.VMEM_SHARED((N,), _f32),
        pltpu.SemaphoreType.DMA,
        pltpu.SemaphoreType.DMA,
        pltpu.SemaphoreType.DMA,
        pltpu.SemaphoreType.DMA,
    ],
)
def _sc_main(x2_hbm, pk2, em2, st_hbm, aggT, deg_out, cs_out, ct_out,
             ds_a, ds_b, si_a, si_b, w_a, w_b, rcs_a, rcs_b,
             rct_a, rct_b, rm, grow_a, grow_b, zrow, zbuf, stv,
             agg_s, deg_s, cs_s, ct_s, gsem_a, gsem_b, ssem_a, ssem_b):
    c = lax.axis_index("c")
    sid = lax.axis_index("s")
    zv = jnp.zeros((16,), _f32)

    def _zb(i, _):
        zbuf[pl.ds(i * 16, 16)] = zv
        return 0
    lax.fori_loop(0, 125, _zb, 0)

    def _zr(i, _):
        for kk in range(8):
            zrow[i, pl.ds(kk * 16, 16)] = zv
        return 0
    lax.fori_loop(0, 16, _zr, 0)
    pltpu.sync_copy(st_hbm, stv)

    @pl.when(sid < 5)
    def _():
        pltpu.sync_copy(zbuf, deg_s.at[pl.ds(sid * 2000, 2000)])

    @pl.when(jnp.logical_and(sid >= 5, sid < 10))
    def _():
        pltpu.sync_copy(zbuf, cs_s.at[pl.ds((sid - 5) * 2000, 2000)])

    @pl.when(jnp.logical_and(sid >= 10, sid < 15))
    def _():
        pltpu.sync_copy(zbuf, ct_s.at[pl.ds((sid - 10) * 2000, 2000)])

    stv_v = stv[...]
    s_vec = _take16(stv_v, jnp.zeros((16,), _i32))
    t_vec = _take16(stv_v, jnp.full((16,), 1, _i32))

    stripe = sid * 640
    zcnt = jnp.where(sid < 15, 40, 26)

    def _bz(q, _):
        pltpu.sync_copy(zrow, agg_s.at[pl.ds(stripe + q * 16, 16)])
        return 0
    lax.fori_loop(0, zcnt, _bz, 0)

    plsc.subcore_barrier()

    def _stage(row, ds_r, si_r, w_r, rcs_r, rct_r, grow_r, gsem):
        pltpu.sync_copy(pk2.at[row], ds_r)
        pltpu.sync_copy(em2.at[row], rm)

        def _grp(k, _):
            m16 = rm[pl.ds(k * 16, 16)]
            d16 = ds_r[pl.ds(k * 16, 16)]
            s16 = ds_r[pl.ds(128 + k * 16, 16)]
            w16 = _sigmoid16(m16)
            w_r[pl.ds(k * 16, 16)] = w16
            si_r[pl.ds(k * 16, 16)] = s16 + s16 + c
            rcs_r[pl.ds(k * 16, 16)] = jnp.where(d16 == s_vec, w16, 0.0)
            rct_r[pl.ds(k * 16, 16)] = jnp.where(d16 == t_vec, w16, 0.0)
            return 0
        lax.fori_loop(0, 8, _grp, 0)
        d_ref = ds_r.at[pl.ds(0, 128)]
        s_ref = ds_r.at[pl.ds(128, 128)]
        hg = pltpu.async_copy(x2_hbm.at[si_r], grow_r, gsem)
        h1 = pltpu.async_copy(w_r, deg_s.at[d_ref], ssem_a, add=True)
        h2 = pltpu.async_copy(rcs_r, cs_s.at[s_ref], ssem_a, add=True)
        h3 = pltpu.async_copy(rct_r, ct_s.at[s_ref], ssem_a, add=True)
        return hg, h1, h2, h3

    def _process(ds_r, w_r, grow_r):
        def _grp(k, _):
            w16 = w_r[pl.ds(k * 16, 16)]
            for r in range(16):
                wr16 = _take16(w16, jnp.full((16,), r, _i32))
                for kk in range(8):
                    grow_r[k * 16 + r, pl.ds(kk * 16, 16)] = (
                        grow_r[k * 16 + r, pl.ds(kk * 16, 16)] * wr16)
            return 0
        lax.fori_loop(0, 8, _grp, 0)
        return pltpu.async_copy(grow_r, agg_s.at[ds_r.at[pl.ds(0, 128)]],
                                ssem_b, add=True)

    def _pair(gp, _):
        row0 = sid + (2 * gp) * NS
        row1 = row0 + NS
        both = row1 < ER

        @pl.when(both)
        def _():
            ha = _stage(row0, ds_a, si_a, w_a, rcs_a, rct_a, grow_a,
                        gsem_a)
            hb = _stage(row1, ds_b, si_b, w_b, rcs_b, rct_b, grow_b,
                        gsem_b)
            ha[0].wait()
            pa = _process(ds_a, w_a, grow_a)
            hb[0].wait()
            pb = _process(ds_b, w_b, grow_b)
            pa.wait()
            pb.wait()
            for h in ha[1:] + hb[1:]:
                h.wait()

        @pl.when(jnp.logical_and(row0 < ER, jnp.logical_not(both)))
        def _():
            ha = _stage(row0, ds_a, si_a, w_a, rcs_a, rct_a, grow_a,
                        gsem_a)
            ha[0].wait()
            pa = _process(ds_a, w_a, grow_a)
            pa.wait()
            for h in ha[1:]:
                h.wait()
        return 0
    lax.fori_loop(0, 40, _pair, 0)

    plsc.subcore_barrier()

    @pl.when(sid < 15)
    def _():
        pltpu.sync_copy(agg_s.at[pl.ds(sid * 632, 632)],
                        aggT.at[c, pl.ds(sid * 632, 632)])

    @pl.when(sid == 15)
    def _():
        pltpu.sync_copy(agg_s.at[pl.ds(9480, 520)],
                        aggT.at[c, pl.ds(9480, 520)])

    @pl.when(jnp.logical_and(c == 0, sid < 5))
    def _():
        pltpu.sync_copy(deg_s.at[pl.ds(sid * 2000, 2000)], zbuf)
        pltpu.sync_copy(zbuf, deg_out.at[pl.ds(sid * 2000, 2000)])

    @pl.when(jnp.logical_and(c == 0, jnp.logical_and(sid >= 5, sid < 10)))
    def _():
        pltpu.sync_copy(cs_s.at[pl.ds((sid - 5) * 2000, 2000)], zbuf)
        pltpu.sync_copy(zbuf, cs_out.at[pl.ds((sid - 5) * 2000, 2000)])

    @pl.when(jnp.logical_and(c == 0, jnp.logical_and(sid >= 10, sid < 15)))
    def _():
        pltpu.sync_copy(ct_s.at[pl.ds((sid - 10) * 2000, 2000)], zbuf)
        pltpu.sync_copy(zbuf, ct_out.at[pl.ds((sid - 10) * 2000, 2000)])


_BN = 1000


def _tc_h1_body(aggT_ref, deg_ref, w1_ref, h1_ref):
    degb = deg_ref[...] + 1e-9
    a0 = aggT_ref[0] / degb
    a1 = aggT_ref[1] / degb
    w1 = w1_ref[...]
    z = (jnp.dot(a0, w1[:128, :], preferred_element_type=_f32,
                 precision=lax.Precision.HIGHEST)
         + jnp.dot(a1, w1[128:, :], preferred_element_type=_f32,
                   precision=lax.Precision.HIGHEST))
    h1_ref[...] = jnp.maximum(z, 0.0)


def _tc_h1(aggT, deg2, W1):
    return pl.pallas_call(
        _tc_h1_body,
        grid=(N // _BN,),
        in_specs=[
            pl.BlockSpec((2, _BN, 128), lambda i: (0, i, 0)),
            pl.BlockSpec((_BN, 1), lambda i: (i, 0)),
            pl.BlockSpec((D, D), lambda i: (0, 0)),
        ],
        out_specs=pl.BlockSpec((_BN, D), lambda i: (i, 0)),
        out_shape=jax.ShapeDtypeStruct((N, D), _f32),
    )(aggT, deg2, W1)


def _tc_final_body(st_ref, cvec_ref, h1_ref, deg_ref, w2_ref, em_ref,
                   out_ref):
    s = st_ref[0, 0]
    t = st_ref[0, 1]
    acc = jnp.dot(cvec_ref[...], h1_ref[...], preferred_element_type=_f32,
                  precision=lax.Precision.HIGHEST)
    deg_s = deg_ref[pl.ds(s, 1), :][0, 0] + 1e-9
    deg_t = deg_ref[pl.ds(t, 1), :][0, 0] + 1e-9
    h2s = jnp.dot((acc[0, :] / deg_s).reshape(1, D), w2_ref[...],
                  preferred_element_type=_f32,
                  precision=lax.Precision.HIGHEST)
    h2t = jnp.dot((acc[1, :] / deg_t).reshape(1, D), w2_ref[...],
                  preferred_element_type=_f32,
                  precision=lax.Precision.HIGHEST)
    score = jnp.sum(h2s * h2t)
    w = jax.nn.sigmoid(em_ref[...])
    eps = 1e-6
    wc = jnp.clip(w, eps, 1.0 - eps)
    ent = -(wc * jnp.log(wc) + (1.0 - wc) * jnp.log(1.0 - wc))
    loss = (-jax.nn.log_sigmoid(score)
            + jnp.sum(w) / E + jnp.sum(ent) / E)
    out_ref[...] = jnp.reshape(loss, (1, 1))


def _tc_final(st2, cvec, h1, deg2, W2, em2):
    return pl.pallas_call(
        _tc_final_body,
        in_specs=[
            pl.BlockSpec(memory_space=pltpu.SMEM),
            pl.BlockSpec((2, N), lambda: (0, 0)),
            pl.BlockSpec((N, D), lambda: (0, 0)),
            pl.BlockSpec((N, 1), lambda: (0, 0)),
            pl.BlockSpec((D, D), lambda: (0, 0)),
            pl.BlockSpec((ER, 128), lambda: (0, 0)),
        ],
        out_specs=pl.BlockSpec((1, 1), lambda: (0, 0)),
        out_shape=jax.ShapeDtypeStruct((1, 1), _f32),
    )(st2, cvec, h1, deg2, W2, em2)


def kernel(x, edge_index, edge_mask, src_nid, tgt_nid, W1, W2):
    src = edge_index[0]
    dst = edge_index[1]
    src2 = src.reshape(ER, 128)
    dst2 = dst.reshape(ER, 128)
    pk2 = jnp.concatenate([dst2, src2], axis=1)
    em2 = edge_mask.reshape(ER, 128)
    st = jnp.zeros((16,), _i32)
    st = st.at[0].set(jnp.asarray(src_nid, _i32))
    st = st.at[1].set(jnp.asarray(tgt_nid, _i32))

    x2 = x.reshape(2 * N, 128)
    aggT, deg, cs, ct = _sc_main(x2, pk2, em2, st)
    cvec = jnp.stack([cs, ct])
    deg2 = deg.reshape(N, 1)
    h1 = _tc_h1(aggT, deg2, W1)
    out = _tc_final(st[:2].reshape(1, 2), cvec, h1, deg2, W2, em2)
    return out[0, 0]
